# Initial kernel scaffold; baseline (speedup 1.0000x reference)
import jax
import jax.numpy as jnp
from jax import lax
from jax.experimental import pallas as pl
from jax.experimental.pallas import tpu as pltpu
from jax.experimental.pallas import tpu_sc as plsc

N_NODES = 10000
N_EDGES = 160000
IN_DIM = 256
HID_DIM = 512
OUT_DIM = 128

NC = 2
NS = 16
NW = NC * NS
EPW = N_EDGES // NW
EB = 125
NBLK = EPW // EB
RPT = N_NODES // NS
ZR = 125

_mesh = plsc.VectorSubcoreMesh(
    core_axis_name="c", subcore_axis_name="s", num_cores=NC, num_subcores=NS)


def _deg_body(dst_hbm, out_hbm, didx, ones, zbuf, acc):
    c = lax.axis_index("c")
    s = lax.axis_index("s")
    wid = s * NC + c

    one16 = jnp.ones((16,), jnp.float32)
    zero16 = jnp.zeros((16,), jnp.float32)

    def fill(i, carry):
        ones[i] = one16
        return carry
    lax.fori_loop(0, EB, fill, 0)

    def fillz(i, carry):
        zbuf[i] = zero16
        return carry
    lax.fori_loop(0, ZR, fillz, 0)

    for k in range(RPT // ZR):
        pltpu.sync_copy(zbuf, acc.at[pl.ds(s * RPT + k * ZR, ZR)])
    plsc.subcore_barrier()

    pltpu.sync_copy(dst_hbm.at[wid], didx)

    def step(j, carry):
        pltpu.sync_copy(ones, acc.at[didx.at[j]], add=True)
        return carry
    lax.fori_loop(0, NBLK, step, 0)
    plsc.subcore_barrier()

    pltpu.sync_copy(acc.at[pl.ds(s * RPT, RPT)],
                    out_hbm.at[pl.ds(c * N_NODES + s * RPT, RPT)])


_deg_call = pl.kernel(
    _deg_body,
    out_type=jax.ShapeDtypeStruct((NC * N_NODES, 16), jnp.float32),
    mesh=_mesh,
    scratch_types=[
        pltpu.VMEM((NBLK, EB), jnp.int32),
        pltpu.VMEM((EB, 16), jnp.float32),
        pltpu.VMEM((ZR, 16), jnp.float32),
        pltpu.VMEM_SHARED((N_NODES, 16), jnp.float32),
    ],
)


def _make_agg(P):
    def body(*refs):
        tabs = refs[:P]
        src_hbm, dst_hbm = refs[P], refs[P + 1]
        out_hbm = refs[P + 2]
        sidx, didx, rows, zbuf, acc = refs[P + 3:P + 8]
        sem = refs[P + 8]

        c = lax.axis_index("c")
        s = lax.axis_index("s")
        wid = s * NC + c

        zero16 = jnp.zeros((16,), jnp.float32)

        def fillz(i, carry):
            for k in range(8):
                zbuf[i, pl.ds(16 * k, 16)] = zero16
            return carry
        lax.fori_loop(0, ZR, fillz, 0)

        pltpu.sync_copy(src_hbm.at[wid], sidx)
        pltpu.sync_copy(dst_hbm.at[wid], didx)

        for p in range(P):
            for k in range(RPT // ZR):
                pltpu.sync_copy(zbuf, acc.at[pl.ds(s * RPT + k * ZR, ZR)])
            plsc.subcore_barrier()

            def step(j, carry):
                pltpu.async_copy(tabs[p].at[sidx.at[j]], rows, sem).wait()
                pltpu.sync_copy(rows, acc.at[didx.at[j]], add=True)
                return carry
            lax.fori_loop(0, NBLK, step, 0)
            plsc.subcore_barrier()

            pltpu.sync_copy(
                acc.at[pl.ds(s * RPT, RPT)],
                out_hbm.at[pl.ds((c * P + p) * N_NODES + s * RPT, RPT)])

    return pl.kernel(
        body,
        out_type=jax.ShapeDtypeStruct((NC * P * N_NODES, 128), jnp.float32),
        mesh=_mesh,
        scratch_types=[
            pltpu.VMEM((NBLK, EB), jnp.int32),
            pltpu.VMEM((NBLK, EB), jnp.int32),
            pltpu.VMEM((EB, 128), jnp.float32),
            pltpu.VMEM((ZR, 128), jnp.float32),
            pltpu.VMEM_SHARED((N_NODES, 128), jnp.float32),
            pltpu.SemaphoreType.DMA,
        ],
    )


_agg2_call = _make_agg(2)
_agg4_call = _make_agg(4)


BN = 1000


def _k0_body(p0, p1, x, xs):
    dinv = lax.rsqrt(1.0 + p0[:, 0:1] + p1[:, 0:1])
    xs[...] = x[...] * dinv


_k0_call = pl.pallas_call(
    _k0_body,
    grid=(N_NODES // BN,),
    in_specs=[
        pl.BlockSpec((BN, 16), lambda i: (i, 0)),
        pl.BlockSpec((BN, 16), lambda i: (i, 0)),
        pl.BlockSpec((BN, IN_DIM), lambda i: (i, 0)),
    ],
    out_specs=pl.BlockSpec((BN, IN_DIM), lambda i: (i, 0)),
    out_shape=jax.ShapeDtypeStruct((N_NODES, IN_DIM), jnp.float32),
)


def _k1_body(y, xs, p0, p1, w1, b1, out):
    dinv = lax.rsqrt(1.0 + p0[:, 0:1] + p1[:, 0:1])
    zl = dinv * (y[0] + y[2] + xs[:, :128])
    zr = dinv * (y[1] + y[3] + xs[:, 128:])
    z = jnp.concatenate([zl, zr], axis=1)
    h = jnp.dot(z, w1[...], preferred_element_type=jnp.float32) + b1[...]
    out[...] = dinv * jnp.maximum(h, 0.0)


_k1_call = pl.pallas_call(
    _k1_body,
    grid=(N_NODES // BN,),
    in_specs=[
        pl.BlockSpec((4, BN, 128), lambda i: (0, i, 0)),
        pl.BlockSpec((BN, IN_DIM), lambda i: (i, 0)),
        pl.BlockSpec((BN, 16), lambda i: (i, 0)),
        pl.BlockSpec((BN, 16), lambda i: (i, 0)),
        pl.BlockSpec((IN_DIM, HID_DIM), lambda i: (0, 0)),
        pl.BlockSpec((1, HID_DIM), lambda i: (0, 0)),
    ],
    out_specs=pl.BlockSpec((BN, HID_DIM), lambda i: (i, 0)),
    out_shape=jax.ShapeDtypeStruct((N_NODES, HID_DIM), jnp.float32),
)


def _k2_body(y, h1s, p0, p1, w2, b2, wfc, bfc, out):
    dinv = lax.rsqrt(1.0 + p0[:, 0:1] + p1[:, 0:1])
    zs = [dinv * (y[p] + y[4 + p] + h1s[:, 128 * p:128 * (p + 1)])
          for p in range(4)]
    z = jnp.concatenate(zs, axis=1)
    h = jnp.dot(z, w2[...], preferred_element_type=jnp.float32) + b2[...]
    h = jnp.maximum(h, 0.0)
    out[...] = jnp.dot(h, wfc[...], preferred_element_type=jnp.float32) + bfc[...]


_k2_call = pl.pallas_call(
    _k2_body,
    grid=(N_NODES // BN,),
    in_specs=[
        pl.BlockSpec((8, BN, 128), lambda i: (0, i, 0)),
        pl.BlockSpec((BN, HID_DIM), lambda i: (i, 0)),
        pl.BlockSpec((BN, 16), lambda i: (i, 0)),
        pl.BlockSpec((BN, 16), lambda i: (i, 0)),
        pl.BlockSpec((HID_DIM, HID_DIM), lambda i: (0, 0)),
        pl.BlockSpec((1, HID_DIM), lambda i: (0, 0)),
        pl.BlockSpec((HID_DIM, OUT_DIM), lambda i: (0, 0)),
        pl.BlockSpec((1, OUT_DIM), lambda i: (0, 0)),
    ],
    out_specs=pl.BlockSpec((BN, OUT_DIM), lambda i: (i, 0)),
    out_shape=jax.ShapeDtypeStruct((N_NODES, OUT_DIM), jnp.float32),
)


def kernel(x, edge_index, output_size, W1, b1, W2, b2, Wfc, bfc):
    ei = edge_index.astype(jnp.int32)
    src3 = ei[0].reshape(NW, NBLK, EB)
    dst3 = ei[1].reshape(NW, NBLK, EB)

    parts = _deg_call(dst3)
    p0, p1 = parts[:N_NODES], parts[N_NODES:]

    xs = _k0_call(p0, p1, x)
    y1 = _agg2_call(xs[:, :128], xs[:, 128:], src3, dst3)
    h1s = _k1_call(y1.reshape(4, N_NODES, 128), xs, p0, p1,
                   W1, b1.reshape(1, HID_DIM))
    y2 = _agg4_call(h1s[:, 0:128], h1s[:, 128:256], h1s[:, 256:384],
                    h1s[:, 384:512], src3, dst3)
    return _k2_call(y2.reshape(8, N_NODES, 128), h1s, p0, p1,
                    W2, b2.reshape(1, HID_DIM), Wfc, bfc.reshape(1, OUT_DIM))


# speedup vs baseline: 12.9649x; 12.9649x over previous
import jax
import jax.numpy as jnp
from jax import lax
from jax.experimental import pallas as pl
from jax.experimental.pallas import tpu as pltpu
from jax.experimental.pallas import tpu_sc as plsc

N_NODES = 10000
N_EDGES = 160000
IN_DIM = 256
HID_DIM = 512
OUT_DIM = 128

NC = 2
NS = 16
NW = NC * NS
EPW = N_EDGES // NW
EB = 125
NBLK = EPW // EB
NPAD = 10240
RPT = NPAD // NS
ZR = 128

_mesh = plsc.VectorSubcoreMesh(
    core_axis_name="c", subcore_axis_name="s", num_cores=NC, num_subcores=NS)


def _deg_body(dst_hbm, out_hbm, didx, ones, zbuf, acc):
    c = lax.axis_index("c")
    s = lax.axis_index("s")
    wid = s * NC + c

    one16 = jnp.ones((16,), jnp.float32)
    zero16 = jnp.zeros((16,), jnp.float32)

    def fill(i, carry):
        ones[i] = one16
        return carry
    lax.fori_loop(0, EB, fill, 0)

    def fillz(i, carry):
        zbuf[i] = zero16
        return carry
    lax.fori_loop(0, ZR, fillz, 0)

    for k in range(RPT // ZR):
        pltpu.sync_copy(zbuf, acc.at[pl.ds(s * RPT + k * ZR, ZR)])
    plsc.subcore_barrier()

    pltpu.sync_copy(dst_hbm.at[wid], didx)

    def step(j, carry):
        pltpu.sync_copy(ones, acc.at[didx.at[j]], add=True)
        return carry
    lax.fori_loop(0, NBLK, step, 0)
    plsc.subcore_barrier()

    pltpu.sync_copy(acc.at[pl.ds(s * RPT, RPT)],
                    out_hbm.at[pl.ds(c * NPAD + s * RPT, RPT)])


_deg_call = pl.kernel(
    _deg_body,
    out_type=jax.ShapeDtypeStruct((NC * NPAD, 16), jnp.float32),
    mesh=_mesh,
    scratch_types=[
        pltpu.VMEM((NBLK, EB), jnp.int32),
        pltpu.VMEM((EB, 16), jnp.float32),
        pltpu.VMEM((ZR, 16), jnp.float32),
        pltpu.VMEM_SHARED((NPAD, 16), jnp.float32),
    ],
)


def _make_agg(P):
    def body(*refs):
        tabs = refs[:P]
        src_hbm, dst_hbm = refs[P], refs[P + 1]
        out_hbm = refs[P + 2]
        sidx, didx, rows, zbuf, acc = refs[P + 3:P + 8]
        sem = refs[P + 8]

        c = lax.axis_index("c")
        s = lax.axis_index("s")
        wid = s * NC + c

        zero16 = jnp.zeros((16,), jnp.float32)

        def fillz(i, carry):
            for k in range(8):
                zbuf[i, pl.ds(16 * k, 16)] = zero16
            return carry
        lax.fori_loop(0, ZR, fillz, 0)

        pltpu.sync_copy(src_hbm.at[wid], sidx)
        pltpu.sync_copy(dst_hbm.at[wid], didx)

        for p in range(P):
            for k in range(RPT // ZR):
                pltpu.sync_copy(zbuf, acc.at[pl.ds(s * RPT + k * ZR, ZR)])
            plsc.subcore_barrier()

            def step(j, carry):
                pltpu.async_copy(tabs[p].at[sidx.at[j]], rows, sem).wait()
                pltpu.sync_copy(rows, acc.at[didx.at[j]], add=True)
                return carry
            lax.fori_loop(0, NBLK, step, 0)
            plsc.subcore_barrier()

            pltpu.sync_copy(
                acc.at[pl.ds(s * RPT, RPT)],
                out_hbm.at[pl.ds((c * P + p) * NPAD + s * RPT, RPT)])

    return pl.kernel(
        body,
        out_type=jax.ShapeDtypeStruct((NC * P * NPAD, 128), jnp.float32),
        mesh=_mesh,
        scratch_types=[
            pltpu.VMEM((NBLK, EB), jnp.int32),
            pltpu.VMEM((NBLK, EB), jnp.int32),
            pltpu.VMEM((EB, 128), jnp.float32),
            pltpu.VMEM((ZR, 128), jnp.float32),
            pltpu.VMEM_SHARED((NPAD, 128), jnp.float32),
            pltpu.SemaphoreType.DMA,
        ],
    )


_agg2_call = _make_agg(2)
_agg4_call = _make_agg(4)


BN = 1000


def _k0_body(p3, x, xs):
    dinv = lax.rsqrt(1.0 + p3[0][:, 0:1] + p3[1][:, 0:1])
    xs[...] = x[...] * dinv


_k0_call = pl.pallas_call(
    _k0_body,
    grid=(N_NODES // BN,),
    in_specs=[
        pl.BlockSpec((2, BN, 16), lambda i: (0, i, 0)),
        pl.BlockSpec((BN, IN_DIM), lambda i: (i, 0)),
    ],
    out_specs=pl.BlockSpec((BN, IN_DIM), lambda i: (i, 0)),
    out_shape=jax.ShapeDtypeStruct((N_NODES, IN_DIM), jnp.float32),
)


def _k1_body(y, xs, p3, w1, b1, out):
    dinv = lax.rsqrt(1.0 + p3[0][:, 0:1] + p3[1][:, 0:1])
    zl = dinv * (y[0] + y[2] + xs[:, :128])
    zr = dinv * (y[1] + y[3] + xs[:, 128:])
    z = jnp.concatenate([zl, zr], axis=1)
    h = jnp.dot(z, w1[...], preferred_element_type=jnp.float32) + b1[...]
    out[...] = dinv * jnp.maximum(h, 0.0)


_k1_call = pl.pallas_call(
    _k1_body,
    grid=(N_NODES // BN,),
    in_specs=[
        pl.BlockSpec((4, BN, 128), lambda i: (0, i, 0)),
        pl.BlockSpec((BN, IN_DIM), lambda i: (i, 0)),
        pl.BlockSpec((2, BN, 16), lambda i: (0, i, 0)),
        pl.BlockSpec((IN_DIM, HID_DIM), lambda i: (0, 0)),
        pl.BlockSpec((1, HID_DIM), lambda i: (0, 0)),
    ],
    out_specs=pl.BlockSpec((BN, HID_DIM), lambda i: (i, 0)),
    out_shape=jax.ShapeDtypeStruct((N_NODES, HID_DIM), jnp.float32),
)


def _k2_body(y, h1s, p3, w2, b2, wfc, bfc, out):
    dinv = lax.rsqrt(1.0 + p3[0][:, 0:1] + p3[1][:, 0:1])
    zs = [dinv * (y[p] + y[4 + p] + h1s[:, 128 * p:128 * (p + 1)])
          for p in range(4)]
    z = jnp.concatenate(zs, axis=1)
    h = jnp.dot(z, w2[...], preferred_element_type=jnp.float32) + b2[...]
    h = jnp.maximum(h, 0.0)
    out[...] = jnp.dot(h, wfc[...], preferred_element_type=jnp.float32) + bfc[...]


_k2_call = pl.pallas_call(
    _k2_body,
    grid=(N_NODES // BN,),
    in_specs=[
        pl.BlockSpec((8, BN, 128), lambda i: (0, i, 0)),
        pl.BlockSpec((BN, HID_DIM), lambda i: (i, 0)),
        pl.BlockSpec((2, BN, 16), lambda i: (0, i, 0)),
        pl.BlockSpec((HID_DIM, HID_DIM), lambda i: (0, 0)),
        pl.BlockSpec((1, HID_DIM), lambda i: (0, 0)),
        pl.BlockSpec((HID_DIM, OUT_DIM), lambda i: (0, 0)),
        pl.BlockSpec((1, OUT_DIM), lambda i: (0, 0)),
    ],
    out_specs=pl.BlockSpec((BN, OUT_DIM), lambda i: (i, 0)),
    out_shape=jax.ShapeDtypeStruct((N_NODES, OUT_DIM), jnp.float32),
)


def kernel(x, edge_index, output_size, W1, b1, W2, b2, Wfc, bfc):
    ei = edge_index.astype(jnp.int32)
    src3 = ei[0].reshape(NW, NBLK, EB)
    dst3 = ei[1].reshape(NW, NBLK, EB)

    p3 = _deg_call(dst3).reshape(NC, NPAD, 16)

    xs = _k0_call(p3, x)
    y1 = _agg2_call(xs[:, :128], xs[:, 128:], src3, dst3)
    h1s = _k1_call(y1.reshape(4, NPAD, 128), xs, p3,
                   W1, b1.reshape(1, HID_DIM))
    y2 = _agg4_call(h1s[:, 0:128], h1s[:, 128:256], h1s[:, 256:384],
                    h1s[:, 384:512], src3, dst3)
    return _k2_call(y2.reshape(8, NPAD, 128), h1s, p3,
                    W2, b2.reshape(1, HID_DIM), Wfc, bfc.reshape(1, OUT_DIM))


# speedup vs baseline: 15.4650x; 1.1928x over previous
import jax
import jax.numpy as jnp
from jax import lax
from jax.experimental import pallas as pl
from jax.experimental.pallas import tpu as pltpu
from jax.experimental.pallas import tpu_sc as plsc

N_NODES = 10000
N_EDGES = 160000
IN_DIM = 256
HID_DIM = 512
OUT_DIM = 128

NC = 2
NS = 16
NW = NC * NS
EPW = N_EDGES // NW
EB = 50
NBLK = EPW // EB
NPAD = 10240
RPT = NPAD // NS
ZR = 128

_mesh = plsc.VectorSubcoreMesh(
    core_axis_name="c", subcore_axis_name="s", num_cores=NC, num_subcores=NS)


def _deg_body(dst_hbm, out_hbm, didx, ones, zbuf, acc):
    c = lax.axis_index("c")
    s = lax.axis_index("s")
    wid = s * NC + c

    one16 = jnp.ones((16,), jnp.float32)
    zero16 = jnp.zeros((16,), jnp.float32)

    def fill(i, carry):
        ones[i] = one16
        return carry
    lax.fori_loop(0, EB, fill, 0)

    def fillz(i, carry):
        zbuf[i] = zero16
        return carry
    lax.fori_loop(0, ZR, fillz, 0)

    for k in range(RPT // ZR):
        pltpu.sync_copy(zbuf, acc.at[pl.ds(s * RPT + k * ZR, ZR)])
    plsc.subcore_barrier()

    pltpu.sync_copy(dst_hbm.at[wid], didx)

    def step(j, carry):
        pltpu.sync_copy(ones, acc.at[didx.at[j]], add=True)
        return carry
    lax.fori_loop(0, NBLK, step, 0)
    plsc.subcore_barrier()

    pltpu.sync_copy(acc.at[pl.ds(s * RPT, RPT)],
                    out_hbm.at[pl.ds(c * NPAD + s * RPT, RPT)])


_deg_call = pl.kernel(
    _deg_body,
    out_type=jax.ShapeDtypeStruct((NC * NPAD, 16), jnp.float32),
    mesh=_mesh,
    scratch_types=[
        pltpu.VMEM((NBLK, EB), jnp.int32),
        pltpu.VMEM((EB, 16), jnp.float32),
        pltpu.VMEM((ZR, 16), jnp.float32),
        pltpu.VMEM_SHARED((NPAD, 16), jnp.float32),
    ],
)


def _make_agg(P):
    def body(*refs):
        tabs = refs[:P]
        src_hbm, dst_hbm = refs[P], refs[P + 1]
        out_hbm = refs[P + 2]
        sidx, didx = refs[P + 3], refs[P + 4]
        rows0, rows1 = refs[P + 5], refs[P + 6]
        acc = refs[P + 7]
        sem0, sem1 = refs[P + 8], refs[P + 9]

        c = lax.axis_index("c")
        s = lax.axis_index("s")
        wid = s * NC + c

        zero16 = jnp.zeros((16,), jnp.float32)

        pltpu.sync_copy(src_hbm.at[wid], sidx)
        pltpu.sync_copy(dst_hbm.at[wid], didx)

        for p in range(P):
            def fillz(i, carry):
                for k in range(8):
                    rows0[i, pl.ds(16 * k, 16)] = zero16
                return carry
            lax.fori_loop(0, EB, fillz, 0)
            for k in range(RPT // EB):
                pltpu.sync_copy(rows0, acc.at[pl.ds(s * RPT + k * EB, EB)])
            pltpu.sync_copy(rows0.at[pl.ds(0, RPT % EB)],
                            acc.at[pl.ds(s * RPT + (RPT // EB) * EB, RPT % EB)])
            plsc.subcore_barrier()

            pltpu.async_copy(tabs[p].at[sidx.at[0]], rows0, sem0)

            def step(i, carry):
                j = 2 * i
                pltpu.async_copy(tabs[p].at[sidx.at[j + 1]], rows1, sem1)
                pltpu.make_async_copy(tabs[p].at[sidx.at[j]], rows0, sem0).wait()
                pltpu.sync_copy(rows0, acc.at[didx.at[j]], add=True)

                @pl.when(j + 2 < NBLK)
                def _():
                    pltpu.async_copy(tabs[p].at[sidx.at[j + 2]], rows0, sem0)
                pltpu.make_async_copy(tabs[p].at[sidx.at[j + 1]], rows1, sem1).wait()
                pltpu.sync_copy(rows1, acc.at[didx.at[j + 1]], add=True)
                return carry
            lax.fori_loop(0, NBLK // 2, step, 0)
            plsc.subcore_barrier()

            pltpu.sync_copy(
                acc.at[pl.ds(s * RPT, RPT)],
                out_hbm.at[pl.ds((c * P + p) * NPAD + s * RPT, RPT)])

    return pl.kernel(
        body,
        out_type=jax.ShapeDtypeStruct((NC * P * NPAD, 128), jnp.float32),
        mesh=_mesh,
        scratch_types=[
            pltpu.VMEM((NBLK, EB), jnp.int32),
            pltpu.VMEM((NBLK, EB), jnp.int32),
            pltpu.VMEM((EB, 128), jnp.float32),
            pltpu.VMEM((EB, 128), jnp.float32),
            pltpu.VMEM_SHARED((NPAD, 128), jnp.float32),
            pltpu.SemaphoreType.DMA,
            pltpu.SemaphoreType.DMA,
        ],
    )


_agg2_call = _make_agg(2)
_agg4_call = _make_agg(4)


BN = 1000


def _k0_body(p3, x, xs0, xs1):
    dinv = lax.rsqrt(1.0 + p3[0][:, 0:1] + p3[1][:, 0:1])
    xs0[...] = x[:, :128] * dinv
    xs1[...] = x[:, 128:] * dinv


_k0_call = pl.pallas_call(
    _k0_body,
    grid=(N_NODES // BN,),
    in_specs=[
        pl.BlockSpec((2, BN, 16), lambda i: (0, i, 0)),
        pl.BlockSpec((BN, IN_DIM), lambda i: (i, 0)),
    ],
    out_specs=[pl.BlockSpec((BN, 128), lambda i: (i, 0))] * 2,
    out_shape=[jax.ShapeDtypeStruct((N_NODES, 128), jnp.float32)] * 2,
)


def _k1_body(y, xs0, xs1, p3, w1, b1, *hs):
    dinv = lax.rsqrt(1.0 + p3[0][:, 0:1] + p3[1][:, 0:1])
    zl = dinv * (y[0] + y[2] + xs0[...])
    zr = dinv * (y[1] + y[3] + xs1[...])
    z = jnp.concatenate([zl, zr], axis=1)
    h = jnp.dot(z, w1[...], preferred_element_type=jnp.float32) + b1[...]
    h = dinv * jnp.maximum(h, 0.0)
    for p in range(4):
        hs[p][...] = h[:, 128 * p:128 * (p + 1)]


_k1_call = pl.pallas_call(
    _k1_body,
    grid=(N_NODES // BN,),
    in_specs=[
        pl.BlockSpec((4, BN, 128), lambda i: (0, i, 0)),
        pl.BlockSpec((BN, 128), lambda i: (i, 0)),
        pl.BlockSpec((BN, 128), lambda i: (i, 0)),
        pl.BlockSpec((2, BN, 16), lambda i: (0, i, 0)),
        pl.BlockSpec((IN_DIM, HID_DIM), lambda i: (0, 0)),
        pl.BlockSpec((1, HID_DIM), lambda i: (0, 0)),
    ],
    out_specs=[pl.BlockSpec((BN, 128), lambda i: (i, 0))] * 4,
    out_shape=[jax.ShapeDtypeStruct((N_NODES, 128), jnp.float32)] * 4,
)


def _k2_body(y, hs0, hs1, hs2, hs3, p3, w2, b2, wfc, bfc, out):
    dinv = lax.rsqrt(1.0 + p3[0][:, 0:1] + p3[1][:, 0:1])
    hs = (hs0, hs1, hs2, hs3)
    zs = [dinv * (y[p] + y[4 + p] + hs[p][...]) for p in range(4)]
    z = jnp.concatenate(zs, axis=1)
    h = jnp.dot(z, w2[...], preferred_element_type=jnp.float32) + b2[...]
    h = jnp.maximum(h, 0.0)
    out[...] = jnp.dot(h, wfc[...], preferred_element_type=jnp.float32) + bfc[...]


_k2_call = pl.pallas_call(
    _k2_body,
    grid=(N_NODES // BN,),
    in_specs=[
        pl.BlockSpec((8, BN, 128), lambda i: (0, i, 0)),
        pl.BlockSpec((BN, 128), lambda i: (i, 0)),
        pl.BlockSpec((BN, 128), lambda i: (i, 0)),
        pl.BlockSpec((BN, 128), lambda i: (i, 0)),
        pl.BlockSpec((BN, 128), lambda i: (i, 0)),
        pl.BlockSpec((2, BN, 16), lambda i: (0, i, 0)),
        pl.BlockSpec((HID_DIM, HID_DIM), lambda i: (0, 0)),
        pl.BlockSpec((1, HID_DIM), lambda i: (0, 0)),
        pl.BlockSpec((HID_DIM, OUT_DIM), lambda i: (0, 0)),
        pl.BlockSpec((1, OUT_DIM), lambda i: (0, 0)),
    ],
    out_specs=pl.BlockSpec((BN, OUT_DIM), lambda i: (i, 0)),
    out_shape=jax.ShapeDtypeStruct((N_NODES, OUT_DIM), jnp.float32),
)


def kernel(x, edge_index, output_size, W1, b1, W2, b2, Wfc, bfc):
    ei = edge_index.astype(jnp.int32)
    src3 = ei[0].reshape(NW, NBLK, EB)
    dst3 = ei[1].reshape(NW, NBLK, EB)

    p3 = _deg_call(dst3).reshape(NC, NPAD, 16)

    xs0, xs1 = _k0_call(p3, x)
    y1 = _agg2_call(xs0, xs1, src3, dst3)
    hs = _k1_call(y1.reshape(4, NPAD, 128), xs0, xs1, p3,
                  W1, b1.reshape(1, HID_DIM))
    y2 = _agg4_call(hs[0], hs[1], hs[2], hs[3], src3, dst3)
    return _k2_call(y2.reshape(8, NPAD, 128), hs[0], hs[1], hs[2], hs[3], p3,
                    W2, b2.reshape(1, HID_DIM), Wfc, bfc.reshape(1, OUT_DIM))
